# baseline (speedup 1.0000x reference)
import jax
import jax.numpy as jnp
from jax import lax
from jax.experimental import pallas as pl
from jax.experimental.pallas import tpu as pltpu
from jax.experimental.pallas import tpu_sc as plsc

NC = 2
NS = 16
NW = NC * NS

KB = 4
PITCH = 129
GROWS = 40
GSTRIDE = GROWS * PITCH


def _emb_body(idx_hbm, table_hbm, out_hbm, idx_v, rows_v, trans_v,
              gsem0, gsem1, osem0, osem1):
    sl = 200
    wid = lax.axis_index("s") * NC + lax.axis_index("c")
    bt0 = wid * KB

    gsems = (gsem0, gsem1)
    osems = (osem0, osem1)

    lane = lax.iota(jnp.int32, 16)
    dt_lo = lane // 8
    dt_hi = dt_lo + 2
    di_vec = lane % 8

    def start_gathers(s, b):
        pltpu.sync_copy(idx_hbm.at[pl.ds(s * 128 + bt0, KB)], idx_v.at[b])
        for k in range(KB):
            for o in range(0, 128, 16):
                i = idx_v[b, k, pl.ds(o, 16)]
                p = (((i >> 16) << 16) | ((i & (TR - 1)) << 2)
                     | ((i >> 14) & 3))
                idx_v[b, k, pl.ds(o, 16)] = p
        for k in range(KB):
            pltpu.async_copy(table_hbm.at[idx_v.at[b, k]],
                             rows_v.at[b, pl.ds(k * 128, 128)], gsems[b])

    def wait_gathers(b):
        for k in range(KB):
            pltpu.make_async_copy(table_hbm.at[idx_v.at[b, k]],
                                  rows_v.at[b, pl.ds(k * 128, 128)],
                                  gsems[b]).wait()

    def transpose(b):
        @plsc.parallel_loop(0, KB * 128, unroll=8)
        def _(bi):
            row_vec = di_vec + jnp.full((16,), (bi >> 7) * 8, jnp.int32)
            bi_vec = jnp.full((16,), bi & 127, jnp.int32)
            lo = rows_v[b, bi, pl.ds(0, 16)]
            hi = rows_v[b, bi, pl.ds(16, 16)]
            plsc.store_scatter(trans_v.at[b], [dt_lo, row_vec, bi_vec], lo)
            plsc.store_scatter(trans_v.at[b], [dt_hi, row_vec, bi_vec], hi)

    def start_writes(s, b):
        for dt in range(4):
            pltpu.async_copy(
                trans3_slice(b, dt),
                out_hbm.at[pl.ds((s * 512 + dt * 128 + bt0) * 8, KB * 8)],
                osems[b])

    def wait_writes(s, b):
        for dt in range(4):
            pltpu.make_async_copy(
                trans3_slice(b, dt),
                out_hbm.at[pl.ds((s * 512 + dt * 128 + bt0) * 8, KB * 8)],
                osems[b]).wait()

    def trans3_slice(b, dt):
        return trans_v.at[b, dt, pl.ds(0, KB * 8), pl.ds(0, 128)]

    start_gathers(0, 0)

    def pair(i, _):
        s0 = i * 2
        s1 = s0 + 1

        start_gathers(s1, 1)
        wait_gathers(0)
        @pl.when(i > 0)
        def _():
            wait_writes(s0 - 2, 0)
        transpose(0)
        start_writes(s0, 0)

        @pl.when(s1 + 1 < sl)
        def _():
            start_gathers(s1 + 1, 0)
        wait_gathers(1)
        @pl.when(i > 0)
        def _():
            wait_writes(s1 - 2, 1)
        transpose(1)
        start_writes(s1, 1)
        return 0

    lax.fori_loop(0, sl // 2, pair, 0)
    wait_writes(sl - 2, 0)
    wait_writes(sl - 1, 1)


TC = 65536
TR = TC // 4


def _tab_body(x_ref, o_ref):
    x = x_ref[...]
    parts = [x[:, g * TR:(g + 1) * TR].T for g in range(4)]
    o_ref[...] = jnp.concatenate(parts, axis=1)


def _linearize_table(tabT):
    dim, nv = tabT.shape
    grid = (nv + TC - 1) // TC
    return pl.pallas_call(
        _tab_body,
        grid=(grid,),
        in_specs=[pl.BlockSpec((dim, TC), lambda i: (0, i))],
        out_specs=pl.BlockSpec((TR, 128), lambda i: (i, 0)),
        out_shape=jax.ShapeDtypeStruct((grid * TR, 128), jnp.float32),
        compiler_params=pltpu.CompilerParams(
            vmem_limit_bytes=128 * 1024 * 1024),
    )(tabT)


def kernel(token_ids, embedding_table):
    nb, sl = token_ids.shape
    dim = embedding_table.shape[1]
    idx = token_ids.T.reshape(sl * (nb // 128), 128)
    tab2 = _linearize_table(embedding_table.T)
    tab = tab2.reshape(tab2.shape[0] * (128 // dim), dim)

    mesh = plsc.VectorSubcoreMesh(core_axis_name="c", subcore_axis_name="s",
                                  num_cores=NC, num_subcores=NS)
    out2 = pl.kernel(
        _emb_body,
        out_type=jax.ShapeDtypeStruct((nb * sl * dim // 128, 128),
                                      jnp.float32),
        mesh=mesh,
        scratch_types=[
            pltpu.VMEM((2, KB, 128), jnp.int32),
            pltpu.VMEM((2, KB * 128, dim), jnp.float32),
            pltpu.VMEM((2, dim // 8, GROWS, PITCH), jnp.float32),
            pltpu.SemaphoreType.DMA,
            pltpu.SemaphoreType.DMA,
            pltpu.SemaphoreType.DMA,
            pltpu.SemaphoreType.DMA,
        ],
        compiler_params=pltpu.CompilerParams(use_tc_tiling_on_sc=False,
                                             needs_layout_passes=False),
    )(idx, tab)
    return (out2.reshape(sl, dim // 8, nb // 128, 8, 128)
            .transpose(2, 4, 0, 1, 3)
            .reshape(nb, sl, dim))


# speedup vs baseline: 1.0085x; 1.0085x over previous
import jax
import jax.numpy as jnp
from jax import lax
from jax.experimental import pallas as pl
from jax.experimental.pallas import tpu as pltpu
from jax.experimental.pallas import tpu_sc as plsc

NC = 2
NS = 16
NW = NC * NS

KB = 4
PITCH = 129
GROWS = 40
GSTRIDE = GROWS * PITCH


def _emb_body(idx_hbm, table_hbm, out_hbm, idx_v, rows_v, trans_v,
              gsem0, gsem1, osem0, osem1):
    sl = 200
    wid = lax.axis_index("s") * NC + lax.axis_index("c")
    bt0 = wid * KB

    gsems = (gsem0, gsem1)
    osems = (osem0, osem1)

    lane = lax.iota(jnp.int32, 16)
    dt_lo = lane // 8
    dt_hi = dt_lo + 2
    di_vec = lane % 8

    def start_gathers(s, b):
        pltpu.sync_copy(idx_hbm.at[pl.ds(s * 128 + bt0, KB)], idx_v.at[b])
        for k in range(KB):
            for o in range(0, 128, 16):
                i = idx_v[b, k, pl.ds(o, 16)]
                p = (((i >> 14) << 14) | ((i & (TR - 1)) << 2)
                     | ((i >> 12) & 3))
                idx_v[b, k, pl.ds(o, 16)] = p
        for k in range(KB):
            pltpu.async_copy(table_hbm.at[idx_v.at[b, k]],
                             rows_v.at[b, pl.ds(k * 128, 128)], gsems[b])

    def wait_gathers(b):
        for k in range(KB):
            pltpu.make_async_copy(table_hbm.at[idx_v.at[b, k]],
                                  rows_v.at[b, pl.ds(k * 128, 128)],
                                  gsems[b]).wait()

    def transpose(b):
        @plsc.parallel_loop(0, KB * 128, unroll=8)
        def _(bi):
            row_vec = di_vec + jnp.full((16,), (bi >> 7) * 8, jnp.int32)
            bi_vec = jnp.full((16,), bi & 127, jnp.int32)
            lo = rows_v[b, bi, pl.ds(0, 16)]
            hi = rows_v[b, bi, pl.ds(16, 16)]
            plsc.store_scatter(trans_v.at[b], [dt_lo, row_vec, bi_vec], lo)
            plsc.store_scatter(trans_v.at[b], [dt_hi, row_vec, bi_vec], hi)

    def start_writes(s, b):
        for dt in range(4):
            pltpu.async_copy(
                trans3_slice(b, dt),
                out_hbm.at[pl.ds((s * 512 + dt * 128 + bt0) * 8, KB * 8)],
                osems[b])

    def wait_writes(s, b):
        for dt in range(4):
            pltpu.make_async_copy(
                trans3_slice(b, dt),
                out_hbm.at[pl.ds((s * 512 + dt * 128 + bt0) * 8, KB * 8)],
                osems[b]).wait()

    def trans3_slice(b, dt):
        return trans_v.at[b, dt, pl.ds(0, KB * 8), pl.ds(0, 128)]

    start_gathers(0, 0)

    def pair(i, _):
        s0 = i * 2
        s1 = s0 + 1

        start_gathers(s1, 1)
        wait_gathers(0)
        @pl.when(i > 0)
        def _():
            wait_writes(s0 - 2, 0)
        transpose(0)
        start_writes(s0, 0)

        @pl.when(s1 + 1 < sl)
        def _():
            start_gathers(s1 + 1, 0)
        wait_gathers(1)
        @pl.when(i > 0)
        def _():
            wait_writes(s1 - 2, 1)
        transpose(1)
        start_writes(s1, 1)
        return 0

    lax.fori_loop(0, sl // 2, pair, 0)
    wait_writes(sl - 2, 0)
    wait_writes(sl - 1, 1)


TC = 16384
TR = TC // 4


def _tab_body(x_ref, o_ref):
    x = x_ref[...]
    parts = [x[:, g * TR:(g + 1) * TR].T for g in range(4)]
    o_ref[...] = jnp.concatenate(parts, axis=1)


def _linearize_table(tabT):
    dim, nv = tabT.shape
    grid = (nv + TC - 1) // TC
    return pl.pallas_call(
        _tab_body,
        grid=(grid,),
        in_specs=[pl.BlockSpec((dim, TC), lambda i: (0, i))],
        out_specs=pl.BlockSpec((TR, 128), lambda i: (i, 0)),
        out_shape=jax.ShapeDtypeStruct((grid * TR, 128), jnp.float32),
    )(tabT)


def kernel(token_ids, embedding_table):
    nb, sl = token_ids.shape
    dim = embedding_table.shape[1]
    idx = token_ids.T.reshape(sl * (nb // 128), 128)
    tab2 = _linearize_table(embedding_table.T)
    tab = tab2.reshape(tab2.shape[0] * (128 // dim), dim)

    mesh = plsc.VectorSubcoreMesh(core_axis_name="c", subcore_axis_name="s",
                                  num_cores=NC, num_subcores=NS)
    out2 = pl.kernel(
        _emb_body,
        out_type=jax.ShapeDtypeStruct((nb * sl * dim // 128, 128),
                                      jnp.float32),
        mesh=mesh,
        scratch_types=[
            pltpu.VMEM((2, KB, 128), jnp.int32),
            pltpu.VMEM((2, KB * 128, dim), jnp.float32),
            pltpu.VMEM((2, dim // 8, GROWS, PITCH), jnp.float32),
            pltpu.SemaphoreType.DMA,
            pltpu.SemaphoreType.DMA,
            pltpu.SemaphoreType.DMA,
            pltpu.SemaphoreType.DMA,
        ],
        compiler_params=pltpu.CompilerParams(use_tc_tiling_on_sc=False,
                                             needs_layout_passes=False),
    )(idx, tab)
    return (out2.reshape(sl, dim // 8, nb // 128, 8, 128)
            .transpose(2, 4, 0, 1, 3)
            .reshape(nb, sl, dim))


# speedup vs baseline: 1.0120x; 1.0035x over previous
import jax
import jax.numpy as jnp
from jax import lax
from jax.experimental import pallas as pl
from jax.experimental.pallas import tpu as pltpu
from jax.experimental.pallas import tpu_sc as plsc

NC = 2
NS = 16
NW = NC * NS

KB = 4
PITCH = 129
GROWS = 40
GSTRIDE = GROWS * PITCH


def _emb_body(idx_hbm, table_hbm, out_hbm, idx_v, rows_v, trans_v,
              gsem0, gsem1, osem0, osem1):
    sl = 200
    wid = lax.axis_index("s") * NC + lax.axis_index("c")
    bt0 = wid * KB

    gsems = (gsem0, gsem1)
    osems = (osem0, osem1)

    lane = lax.iota(jnp.int32, 16)
    dt_lo = lane // 8
    dt_hi = dt_lo + 2
    di_vec = lane % 8

    def start_gathers(s, b):
        pltpu.sync_copy(idx_hbm.at[pl.ds(s * 128 + bt0, KB)], idx_v.at[b])
        for k in range(KB):
            for o in range(0, 128, 16):
                i = idx_v[b, k, pl.ds(o, 16)]
                p = (((i >> 15) << 15) | ((i & (TR - 1)) << 2)
                     | ((i >> 13) & 3))
                idx_v[b, k, pl.ds(o, 16)] = p
        for k in range(KB):
            pltpu.async_copy(table_hbm.at[idx_v.at[b, k]],
                             rows_v.at[b, pl.ds(k * 128, 128)], gsems[b])

    def wait_gathers(b):
        for k in range(KB):
            pltpu.make_async_copy(table_hbm.at[idx_v.at[b, k]],
                                  rows_v.at[b, pl.ds(k * 128, 128)],
                                  gsems[b]).wait()

    def transpose(b):
        @plsc.parallel_loop(0, KB * 128, unroll=8)
        def _(bi):
            row_vec = di_vec + jnp.full((16,), (bi >> 7) * 8, jnp.int32)
            bi_vec = jnp.full((16,), bi & 127, jnp.int32)
            lo = rows_v[b, bi, pl.ds(0, 16)]
            hi = rows_v[b, bi, pl.ds(16, 16)]
            plsc.store_scatter(trans_v.at[b], [dt_lo, row_vec, bi_vec], lo)
            plsc.store_scatter(trans_v.at[b], [dt_hi, row_vec, bi_vec], hi)

    def start_writes(s, b):
        for dt in range(4):
            pltpu.async_copy(
                trans3_slice(b, dt),
                out_hbm.at[pl.ds((s * 512 + dt * 128 + bt0) * 8, KB * 8)],
                osems[b])

    def wait_writes(s, b):
        for dt in range(4):
            pltpu.make_async_copy(
                trans3_slice(b, dt),
                out_hbm.at[pl.ds((s * 512 + dt * 128 + bt0) * 8, KB * 8)],
                osems[b]).wait()

    def trans3_slice(b, dt):
        return trans_v.at[b, dt, pl.ds(0, KB * 8), pl.ds(0, 128)]

    start_gathers(0, 0)

    def pair(i, _):
        s0 = i * 2
        s1 = s0 + 1

        start_gathers(s1, 1)
        wait_gathers(0)
        @pl.when(i > 0)
        def _():
            wait_writes(s0 - 2, 0)
        transpose(0)
        start_writes(s0, 0)

        @pl.when(s1 + 1 < sl)
        def _():
            start_gathers(s1 + 1, 0)
        wait_gathers(1)
        @pl.when(i > 0)
        def _():
            wait_writes(s1 - 2, 1)
        transpose(1)
        start_writes(s1, 1)
        return 0

    lax.fori_loop(0, sl // 2, pair, 0)
    wait_writes(sl - 2, 0)
    wait_writes(sl - 1, 1)


TC = 32768
TR = TC // 4


def _tab_body(x_ref, o_ref):
    for g in range(4):
        o_ref[:, g * 32:(g + 1) * 32] = x_ref[:, g * TR:(g + 1) * TR].T


def _linearize_table(tabT):
    dim, nv = tabT.shape
    grid = (nv + TC - 1) // TC
    return pl.pallas_call(
        _tab_body,
        grid=(grid,),
        in_specs=[pl.BlockSpec((dim, TC), lambda i: (0, i))],
        out_specs=pl.BlockSpec((TR, 128), lambda i: (i, 0)),
        out_shape=jax.ShapeDtypeStruct((grid * TR, 128), jnp.float32),
    )(tabT)


def kernel(token_ids, embedding_table):
    nb, sl = token_ids.shape
    dim = embedding_table.shape[1]
    idx = token_ids.T.reshape(sl * (nb // 128), 128)
    tab2 = _linearize_table(embedding_table.T)
    tab = tab2.reshape(tab2.shape[0] * (128 // dim), dim)

    mesh = plsc.VectorSubcoreMesh(core_axis_name="c", subcore_axis_name="s",
                                  num_cores=NC, num_subcores=NS)
    out2 = pl.kernel(
        _emb_body,
        out_type=jax.ShapeDtypeStruct((nb * sl * dim // 128, 128),
                                      jnp.float32),
        mesh=mesh,
        scratch_types=[
            pltpu.VMEM((2, KB, 128), jnp.int32),
            pltpu.VMEM((2, KB * 128, dim), jnp.float32),
            pltpu.VMEM((2, dim // 8, GROWS, PITCH), jnp.float32),
            pltpu.SemaphoreType.DMA,
            pltpu.SemaphoreType.DMA,
            pltpu.SemaphoreType.DMA,
            pltpu.SemaphoreType.DMA,
        ],
        compiler_params=pltpu.CompilerParams(use_tc_tiling_on_sc=False,
                                             needs_layout_passes=False),
    )(idx, tab)
    return (out2.reshape(sl, dim // 8, nb // 128, 8, 128)
            .transpose(2, 4, 0, 1, 3)
            .reshape(nb, sl, dim))
